# baseline (speedup 1.0000x reference)
import functools
import jax
import jax.numpy as jnp
from jax import lax
from jax.experimental import pallas as pl
from jax.experimental.pallas import tpu as pltpu
from jax.experimental.pallas import tpu_sc as plsc

_L = 16
_C = 80
_NB = 5


@functools.lru_cache(maxsize=None)
def _build(B, V, D, dtype_name):
    info = plsc.get_sparse_core_info()
    NC, NS = info.num_cores, info.num_subcores
    NW = NC * NS
    C, NB = _C, _NB
    assert B % (NW * C) == 0
    G = B // (NW * C)
    assert G % NB == 0
    T = G // NB
    b_per_w = B // NW
    dtype = jnp.dtype(dtype_name)

    mesh = plsc.VectorSubcoreMesh(core_axis_name="c", subcore_axis_name="s")

    @functools.partial(
        pl.kernel,
        mesh=mesh,
        out_type=jax.ShapeDtypeStruct((B, D), dtype),
        scratch_types=(
            [pltpu.VMEM((G, C), jnp.int32)]
            + [pltpu.VMEM((C, D), dtype) for _ in range(2 * NB)]
            + [pltpu.VMEM((_L,), dtype)]
            + [pltpu.SemaphoreType.DMA for _ in range(2 * NB)]
        ),
    )
    def emb_kernel(idx_hbm, table_hbm, scale_hbm, out_hbm, idx_v, *rest):
        in_bufs = rest[0:NB]
        out_bufs = rest[NB:2 * NB]
        scl_v = rest[2 * NB]
        gi = rest[2 * NB + 1:2 * NB + 1 + NB]
        go = rest[2 * NB + 1 + NB:2 * NB + 1 + 2 * NB]

        wid = lax.axis_index("s") * NC + lax.axis_index("c")
        base = wid * b_per_w
        pltpu.sync_copy(idx_hbm.at[wid], idx_v)
        pltpu.sync_copy(scale_hbm, scl_v)
        s = scl_v[...]

        def gather_start(g, buf, sem):
            pltpu.async_copy(table_hbm.at[idx_v.at[g]], buf, sem)

        def gather_wait(buf, sem):
            pltpu.make_async_copy(table_hbm.at[pl.ds(0, C)], buf, sem).wait()

        def out_start(g, buf, sem):
            pltpu.async_copy(buf, out_hbm.at[pl.ds(base + g * C, C)], sem)

        def out_wait(buf, sem):
            pltpu.make_async_copy(buf, out_hbm.at[pl.ds(0, C)], sem).wait()

        def scale(src, dst):
            @plsc.parallel_loop(0, C, step=1, unroll=4)
            def _row(r):
                for j in range(D // _L):
                    sl = pl.ds(j * _L, _L)
                    dst[r, sl] = src[r, sl] * s

        for k in range(NB):
            gather_start(k, in_bufs[k], gi[k])

        def body(t, carry):
            for k in range(NB):
                g = NB * t + k
                gather_wait(in_bufs[k], gi[k])

                @pl.when(t > 0)
                def _():
                    out_wait(out_bufs[k], go[k])

                scale(in_bufs[k], out_bufs[k])
                out_start(g, out_bufs[k], go[k])

                @pl.when(t + 1 < T)
                def _():
                    gather_start(g + NB, in_bufs[k], gi[k])

            return carry

        lax.fori_loop(0, T, body, 0)
        for k in range(NB):
            out_wait(out_bufs[k], go[k])

    return emb_kernel, NW, C, G


def kernel(x, table, scale):
    Bt, S = x.shape
    V, D = table.shape
    B = Bt * S
    emb_kernel, NW, C, G = _build(B, V, D, table.dtype.name)
    idx3 = x.reshape(NW, G, C)
    scale_v = jnp.full((_L,), scale, dtype=table.dtype)
    out = emb_kernel(idx3, table, scale_v)
    return out.reshape(Bt, S, D)


# speedup vs baseline: 1.0045x; 1.0045x over previous
import functools
import jax
import jax.numpy as jnp
from jax import lax
from jax.experimental import pallas as pl
from jax.experimental.pallas import tpu as pltpu
from jax.experimental.pallas import tpu_sc as plsc

_L = 16
_C = 80
_NB = 5


@functools.lru_cache(maxsize=None)
def _build(B, V, D, dtype_name):
    info = plsc.get_sparse_core_info()
    NC, NS = info.num_cores, info.num_subcores
    NW = NC * NS
    C, NB = _C, _NB
    assert B % (NW * C) == 0
    G = B // (NW * C)
    assert G % NB == 0
    T = G // NB
    b_per_w = B // NW
    dtype = jnp.dtype(dtype_name)

    mesh = plsc.VectorSubcoreMesh(core_axis_name="c", subcore_axis_name="s")

    @functools.partial(
        pl.kernel,
        mesh=mesh,
        out_type=jax.ShapeDtypeStruct((B, D), dtype),
        scratch_types=(
            [pltpu.VMEM((G * C,), jnp.int32)]
            + [pltpu.VMEM((C, D), dtype) for _ in range(2 * NB)]
            + [pltpu.VMEM((_L,), dtype)]
            + [pltpu.SemaphoreType.DMA for _ in range(2 * NB)]
        ),
    )
    def emb_kernel(idx_hbm, table_hbm, scale_hbm, out_hbm, idx_v, *rest):
        in_bufs = rest[0:NB]
        out_bufs = rest[NB:2 * NB]
        scl_v = rest[2 * NB]
        gi = rest[2 * NB + 1:2 * NB + 1 + NB]
        go = rest[2 * NB + 1 + NB:2 * NB + 1 + 2 * NB]

        wid = lax.axis_index("s") * NC + lax.axis_index("c")
        base = wid * b_per_w
        pltpu.sync_copy(idx_hbm.at[pl.ds(base, b_per_w)], idx_v)
        pltpu.sync_copy(scale_hbm, scl_v)
        s = scl_v[...]

        def gather_start(g, buf, sem):
            pltpu.async_copy(table_hbm.at[idx_v.at[pl.ds(g * C, C)]], buf, sem)

        def gather_wait(buf, sem):
            pltpu.make_async_copy(table_hbm.at[pl.ds(0, C)], buf, sem).wait()

        def out_start(g, buf, sem):
            pltpu.async_copy(buf, out_hbm.at[pl.ds(base + g * C, C)], sem)

        def out_wait(buf, sem):
            pltpu.make_async_copy(buf, out_hbm.at[pl.ds(0, C)], sem).wait()

        def scale(src, dst):
            @plsc.parallel_loop(0, C, step=1, unroll=4)
            def _row(r):
                for j in range(D // _L):
                    sl = pl.ds(j * _L, _L)
                    dst[r, sl] = src[r, sl] * s

        for k in range(NB):
            gather_start(k, in_bufs[k], gi[k])

        def body(t, carry):
            for k in range(NB):
                g = NB * t + k
                gather_wait(in_bufs[k], gi[k])

                @pl.when(t > 0)
                def _():
                    out_wait(out_bufs[k], go[k])

                scale(in_bufs[k], out_bufs[k])
                out_start(g, out_bufs[k], go[k])

                @pl.when(t + 1 < T)
                def _():
                    gather_start(g + NB, in_bufs[k], gi[k])

            return carry

        lax.fori_loop(0, T, body, 0)
        for k in range(NB):
            out_wait(out_bufs[k], go[k])

    return emb_kernel, NW, C, G


def kernel(x, table, scale):
    Bt, S = x.shape
    V, D = table.shape
    B = Bt * S
    emb_kernel, NW, C, G = _build(B, V, D, table.dtype.name)
    idx_flat = x.reshape(B)
    scale_v = jnp.full((_L,), scale, dtype=table.dtype)
    out = emb_kernel(idx_flat, table, scale_v)
    return out.reshape(Bt, S, D)


# speedup vs baseline: 1.0140x; 1.0095x over previous
import functools
import jax
import jax.numpy as jnp
from jax import lax
from jax.experimental import pallas as pl
from jax.experimental.pallas import tpu as pltpu
from jax.experimental.pallas import tpu_sc as plsc

_L = 16
_C = 80
_NB = 5


@functools.lru_cache(maxsize=None)
def _build(B, V, D, dtype_name):
    info = plsc.get_sparse_core_info()
    NC, NS = info.num_cores, info.num_subcores
    NW = NC * NS
    C, NB = _C, _NB
    assert B % (NW * C) == 0
    G = B // (NW * C)
    assert G % NB == 0
    T = G // NB
    b_per_w = B // NW
    dtype = jnp.dtype(dtype_name)

    mesh = plsc.VectorSubcoreMesh(core_axis_name="c", subcore_axis_name="s")

    @functools.partial(
        pl.kernel,
        mesh=mesh,
        out_type=jax.ShapeDtypeStruct((B, D), dtype),
        scratch_types=(
            [pltpu.VMEM((G * C,), jnp.int32)]
            + [pltpu.VMEM((C, D), dtype) for _ in range(2 * NB)]
            + [pltpu.VMEM((_L,), dtype)]
            + [pltpu.SemaphoreType.DMA for _ in range(2 * NB)]
        ),
    )
    def emb_kernel(idx_hbm, table_hbm, scale_hbm, out_hbm, idx_v, *rest):
        in_bufs = rest[0:NB]
        out_bufs = rest[NB:2 * NB]
        scl_v = rest[2 * NB]
        gi = rest[2 * NB + 1:2 * NB + 1 + NB]
        go = rest[2 * NB + 1 + NB:2 * NB + 1 + 2 * NB]

        wid = lax.axis_index("s") * NC + lax.axis_index("c")
        base = wid * b_per_w
        scl_cp = pltpu.async_copy(scale_hbm, scl_v, go[0])
        idx_cp = pltpu.async_copy(idx_hbm.at[pl.ds(base, b_per_w)], idx_v, gi[0])
        idx_cp.wait()

        def gather_start(g, buf, sem):
            pltpu.async_copy(table_hbm.at[idx_v.at[pl.ds(g * C, C)]], buf, sem)

        def gather_wait(buf, sem):
            pltpu.make_async_copy(table_hbm.at[pl.ds(0, C)], buf, sem).wait()

        def out_start(g, buf, sem):
            pltpu.async_copy(buf, out_hbm.at[pl.ds(base + g * C, C)], sem)

        def out_wait(buf, sem):
            pltpu.make_async_copy(buf, out_hbm.at[pl.ds(0, C)], sem).wait()

        def scale(src, dst):
            @plsc.parallel_loop(0, C, step=1, unroll=4)
            def _row(r):
                for j in range(D // _L):
                    sl = pl.ds(j * _L, _L)
                    dst[r, sl] = src[r, sl] * s

        for k in range(NB):
            gather_start(k, in_bufs[k], gi[k])
        scl_cp.wait()
        s = scl_v[...]

        def body(t, carry):
            for k in range(NB):
                g = NB * t + k
                gather_wait(in_bufs[k], gi[k])

                @pl.when(t > 0)
                def _():
                    out_wait(out_bufs[k], go[k])

                scale(in_bufs[k], out_bufs[k])
                out_start(g, out_bufs[k], go[k])

                @pl.when(t + 1 < T)
                def _():
                    gather_start(g + NB, in_bufs[k], gi[k])

            return carry

        lax.fori_loop(0, T, body, 0)
        for k in range(NB):
            out_wait(out_bufs[k], go[k])

    return emb_kernel, NW, C, G


def kernel(x, table, scale):
    Bt, S = x.shape
    V, D = table.shape
    B = Bt * S
    emb_kernel, NW, C, G = _build(B, V, D, table.dtype.name)
    idx_flat = x.reshape(B)
    scale_v = jnp.full((_L,), scale, dtype=table.dtype)
    out = emb_kernel(idx_flat, table, scale_v)
    return out.reshape(Bt, S, D)
